# baseline (speedup 1.0000x reference)
import jax
import jax.numpy as jnp
from jax.experimental import pallas as pl
from jax.experimental.pallas import tpu as pltpu

B, L, D, H, O, TOPK = 8, 2048, 128, 128, 64, 64
_SCALE = float(H) ** 0.5
_INV_SCALE = 1.0 / _SCALE
_BIG = 1 << 30
_TR = 256


def _fused_body(x_ref, xt_ref, pw1, pb1, pw2, pb2, qw, qb, kw, kb,
                xw1, xb1, xw2, xb2, rw1, rb1, rw2, rb2, out_ref,
                q_ref, k_ref, rm_ref, cc_ref, dc_ref, m2_ref, c2_ref,
                vals_ref, pairs_ref):
    x2 = x_ref[0]
    xt = xt_ref[0]
    neg_inf = jnp.float32(-jnp.inf)

    colabs = jnp.sum(jnp.abs(xt), axis=0, keepdims=True)
    validc = colabs != 0.0
    validf = validc.astype(jnp.float32)
    count = jnp.sum(validf)
    rowabs = jnp.sum(jnp.abs(x2), axis=1, keepdims=True)
    validr = rowabs != 0.0

    h = jnp.maximum(jnp.dot(x2, pw1[...]) + pb1[...], 0.0)
    phi_x = jnp.dot(h, pw2[...]) + pb2[...]
    phi_pooled = jnp.dot(validf, phi_x) / jnp.maximum(count, 1.0)

    q = jnp.dot(x2, qw[...]) + qb[...]
    k = jnp.dot(x2, kw[...]) + kb[...]
    q_ref[...] = q
    k_ref[...] = k

    ci_t = jax.lax.broadcasted_iota(jnp.int32, (_TR, L), 1)
    ri_t = jax.lax.broadcasted_iota(jnp.int32, (_TR, L), 0)
    for t in range(L // _TR):
        st = jax.lax.dot_general(q[t * _TR:(t + 1) * _TR], k,
                                 (((1,), (1,)), ((), ()))) * _INV_SCALE
        okt = (validr[t * _TR:(t + 1) * _TR] & validc
               & ((ri_t + t * _TR) != ci_t))
        smt = jnp.where(okt, st, neg_inf)
        m1 = jnp.max(smt, axis=1)
        c1 = jnp.min(jnp.where(smt == m1[:, None], ci_t, _BIG), axis=1)
        sm2 = jnp.where(ci_t == c1[:, None], neg_inf, smt)
        m2 = jnp.max(sm2, axis=1)
        c2 = jnp.min(jnp.where(sm2 == m2[:, None], ci_t, _BIG), axis=1)
        rs = _TR // 128
        rm_ref[rs * t:rs * (t + 1), :] = m1.reshape(rs, 128)
        cc_ref[rs * t:rs * (t + 1), :] = jnp.minimum(c1, L - 1).reshape(rs, 128)
        m2_ref[rs * t:rs * (t + 1), :] = m2.reshape(rs, 128)
        c2_ref[rs * t:rs * (t + 1), :] = jnp.minimum(c2, L - 1).reshape(rs, 128)
        dc_ref[rs * t:rs * (t + 1), :] = jnp.zeros((rs, 128), jnp.int32)

    fi = (jax.lax.broadcasted_iota(jnp.int32, (16, 128), 0) * 128
          + jax.lax.broadcasted_iota(jnp.int32, (16, 128), 1))
    li1 = jax.lax.broadcasted_iota(jnp.int32, (1, L), 1)
    sv = jax.lax.broadcasted_iota(jnp.int32, (8, 128), 0)
    lv = jax.lax.broadcasted_iota(jnp.int32, (8, 128), 1)
    sp = jax.lax.broadcasted_iota(jnp.int32, (TOPK, 2 * D), 0)

    def body(t, carry):
        rm = rm_ref[...]
        m = jnp.max(rm)
        r = jnp.min(jnp.where(rm == m, fi, _BIG))
        onr = fi == r
        c = jnp.min(jnp.where(onr, cc_ref[...], _BIG))
        d = jnp.min(jnp.where(onr, dc_ref[...], _BIG))

        vals_ref[...] = jnp.where((sv == 0) & (lv == t), m, vals_ref[...])
        xr = x_ref[0, pl.ds(r, 1), :]
        xc = x_ref[0, pl.ds(c, 1), :]
        pair_row = jnp.concatenate([xr, xc], axis=1)
        pairs_ref[...] = jnp.where(sp == t, pair_row, pairs_ref[...])

        d1 = d + 1
        nm_cheap = jnp.max(jnp.where(onr, m2_ref[...], neg_inf))
        nc_cheap = jnp.min(jnp.where(onr, c2_ref[...], _BIG))

        def rare(_):
            qr = q_ref[pl.ds(r, 1), :]
            srow = jax.lax.dot_general(qr, k_ref[...],
                                       (((1,), (1,)), ((), ()))) * _INV_SCALE
            srow = jnp.where(validc & (li1 != r), srow, neg_inf)

            def ext(j, st):
                row, _, _ = st
                mj = jnp.max(row)
                cj = jnp.min(jnp.where(row == mj, li1, _BIG))
                return (jnp.where(li1 == cj, neg_inf, row), mj, cj)

            _, mj, cj = jax.lax.fori_loop(0, d1 + 1, ext,
                                          (srow, neg_inf, jnp.int32(0)))
            return mj, jnp.minimum(cj, L - 1)

        def cheap(_):
            return nm_cheap, jnp.minimum(nc_cheap, L - 1)

        val_n, col_n = jax.lax.cond(d1 >= 2, rare, cheap, 0)
        rm_ref[...] = jnp.where(onr, val_n, rm)
        cc_ref[...] = jnp.where(onr, col_n, cc_ref[...])
        dc_ref[...] = jnp.where(onr, d1, dc_ref[...])
        return carry

    jax.lax.fori_loop(0, TOPK, body, 0)

    vals = vals_ref[0:1, 0:TOPK]
    mv = jnp.max(vals)
    e = jnp.exp(vals - mv)
    w = e / jnp.sum(e)

    pairs = pairs_ref[...]
    h1 = jnp.maximum(jnp.dot(pairs, xw1[...]) + xb1[...], 0.0)
    xi_x = jnp.dot(h1, xw2[...]) + xb2[...]
    xi_pooled = jnp.dot(w, xi_x)

    pooled = jnp.concatenate([phi_pooled, xi_pooled], axis=1)
    h2 = jnp.maximum(jnp.dot(pooled, rw1[...]) + rb1[...], 0.0)
    out_ref[0] = jnp.dot(h2, rw2[...]) + rb2[...]


def kernel(x, phi_W1, phi_b1, phi_W2, phi_b2, q_W, q_b, k_W, k_b,
           xi_W1, xi_b1, xi_W2, xi_b2, rho_W1, rho_b1, rho_W2, rho_b2):
    xt = jnp.swapaxes(x, 1, 2)

    def wspec(shape):
        return pl.BlockSpec(shape, lambda b: (0,) * len(shape))

    weights = [
        (phi_W1.T, wspec((D, H))), (phi_b1.reshape(1, H), wspec((1, H))),
        (phi_W2.T, wspec((H, H))), (phi_b2.reshape(1, H), wspec((1, H))),
        (q_W.T, wspec((D, H))), (q_b.reshape(1, H), wspec((1, H))),
        (k_W.T, wspec((D, H))), (k_b.reshape(1, H), wspec((1, H))),
        (xi_W1.T, wspec((2 * D, H))), (xi_b1.reshape(1, H), wspec((1, H))),
        (xi_W2.T, wspec((H, H))), (xi_b2.reshape(1, H), wspec((1, H))),
        (rho_W1.T, wspec((2 * H, H))), (rho_b1.reshape(1, H), wspec((1, H))),
        (rho_W2.T, wspec((H, O))), (rho_b2.reshape(1, O), wspec((1, O))),
    ]

    out = pl.pallas_call(
        _fused_body,
        grid=(B,),
        in_specs=[
            pl.BlockSpec((1, L, D), lambda b: (b, 0, 0)),
            pl.BlockSpec((1, D, L), lambda b: (b, 0, 0)),
        ] + [spec for _, spec in weights],
        out_specs=pl.BlockSpec((1, 1, O), lambda b: (b, 0, 0)),
        out_shape=jax.ShapeDtypeStruct((B, 1, O), jnp.float32),
        scratch_shapes=[
            pltpu.VMEM((L, D), jnp.float32),
            pltpu.VMEM((L, D), jnp.float32),
            pltpu.VMEM((16, 128), jnp.float32),
            pltpu.VMEM((16, 128), jnp.int32),
            pltpu.VMEM((16, 128), jnp.int32),
            pltpu.VMEM((16, 128), jnp.float32),
            pltpu.VMEM((16, 128), jnp.int32),
            pltpu.VMEM((8, 128), jnp.float32),
            pltpu.VMEM((TOPK, 2 * D), jnp.float32),
        ],
        compiler_params=pltpu.CompilerParams(
            dimension_semantics=("arbitrary",),
        ),
    )(x, xt, *[w for w, _ in weights])
    return out.reshape(B, O)
